# baseline (speedup 1.0000x reference)
import functools

import numpy as np
import jax
import jax.numpy as jnp
from jax.experimental import pallas as pl
from jax.experimental.pallas import tpu as pltpu


@functools.lru_cache(maxsize=None)
def _filter_consts(H: int, W: int, rate: float):
    n = np.arange(W)
    ang = 2.0 * np.pi * np.outer(n, n) / W
    Wc = np.exp(-1j * ang)
    Vc = np.exp(+1j * ang) / W
    cy, cx = H // 2, W // 2
    rh, rw = int(rate * cy), int(rate * cx)
    cols = np.arange(W)
    col_keep = (~((cols >= cx - rw) & (cols < cx + rw))).astype(np.float64)
    A = np.real((Wc * col_keep[None, :]) @ Vc).astype(np.float32)
    return A, cy - rh, cy + rh


def _row_filter_body(x_ref, a_ref, m_ref, o_ref):
    x = x_ref[...]
    y = jnp.dot(x.astype(jnp.bfloat16), a_ref[...],
                preferred_element_type=jnp.float32)
    o_ref[...] = jnp.where(m_ref[...] != 0, y, x)


def kernel(x, rate: float = 0.95):
    B, C, H, W = x.shape
    A_np, lo, hi = _filter_consts(int(H), int(W), float(rate))
    A = jnp.asarray(A_np, dtype=jnp.bfloat16)

    M = B * C * H
    xf = x.reshape(M, W).astype(jnp.float32)

    TM = 8192
    while M % TM != 0 or TM % H != 0:
        TM //= 2

    r = np.arange(TM) % H
    mask = jnp.asarray(((r >= lo) & (r < hi)).reshape(TM, 1)
                       .astype(np.float32))

    out = pl.pallas_call(
        _row_filter_body,
        out_shape=jax.ShapeDtypeStruct((M, W), jnp.float32),
        grid=(M // TM,),
        in_specs=[
            pl.BlockSpec((TM, W), lambda i: (i, 0)),
            pl.BlockSpec((W, W), lambda i: (0, 0)),
            pl.BlockSpec((TM, 1), lambda i: (0, 0)),
        ],
        out_specs=pl.BlockSpec((TM, W), lambda i: (i, 0)),
        compiler_params=pltpu.CompilerParams(
            dimension_semantics=("arbitrary",),
            vmem_limit_bytes=64 * 2 ** 20),
    )(xf, A, mask)

    return out.reshape(B, C, H, W)


# speedup vs baseline: 1.1107x; 1.1107x over previous
import functools

import numpy as np
import jax
import jax.numpy as jnp
from jax.experimental import pallas as pl
from jax.experimental.pallas import tpu as pltpu


@functools.lru_cache(maxsize=None)
def _filter_consts(H: int, W: int, rate: float):
    n = np.arange(W)
    ang = 2.0 * np.pi * np.outer(n, n) / W
    Wc = np.exp(-1j * ang)
    Vc = np.exp(+1j * ang) / W
    cy, cx = H // 2, W // 2
    rh, rw = int(rate * cy), int(rate * cx)
    cols = np.arange(W)
    col_keep = (~((cols >= cx - rw) & (cols < cx + rw))).astype(np.float64)
    A = np.real((Wc * col_keep[None, :]) @ Vc).astype(np.float32)
    return A, cy - rh, cy + rh


def _row_filter_body(TM, H, lo, hi, x_ref, a_ref, o_ref):
    x = x_ref[...]
    y = jnp.dot(x.astype(jnp.bfloat16), a_ref[...],
                preferred_element_type=jnp.float32)
    o_ref[...] = y
    for base in range(0, TM, H):
        if lo > 0:
            o_ref[base:base + lo, :] = x[base:base + lo, :]
        if hi < H:
            o_ref[base + hi:base + H, :] = x[base + hi:base + H, :]


def kernel(x, rate: float = 0.95):
    B, C, H, W = x.shape
    A_np, lo, hi = _filter_consts(int(H), int(W), float(rate))
    A = jnp.asarray(A_np, dtype=jnp.bfloat16)

    M = B * C * H
    xf = x.reshape(M, W).astype(jnp.float32)

    TM = 8192
    while M % TM != 0 or TM % H != 0:
        TM //= 2

    out = pl.pallas_call(
        functools.partial(_row_filter_body, TM, H, lo, hi),
        out_shape=jax.ShapeDtypeStruct((M, W), jnp.float32),
        grid=(M // TM,),
        in_specs=[
            pl.BlockSpec((TM, W), lambda i: (i, 0)),
            pl.BlockSpec((W, W), lambda i: (0, 0)),
        ],
        out_specs=pl.BlockSpec((TM, W), lambda i: (i, 0)),
        compiler_params=pltpu.CompilerParams(
            dimension_semantics=("arbitrary",),
            vmem_limit_bytes=64 * 2 ** 20),
    )(xf, A)

    return out.reshape(B, C, H, W)


# speedup vs baseline: 1.1700x; 1.0534x over previous
import functools

import numpy as np
import jax
import jax.numpy as jnp
from jax.experimental import pallas as pl
from jax.experimental.pallas import tpu as pltpu


@functools.lru_cache(maxsize=None)
def _filter_consts(H: int, W: int, rate: float):
    n = np.arange(W)
    ang = 2.0 * np.pi * np.outer(n, n) / W
    Wc = np.exp(-1j * ang)
    Vc = np.exp(+1j * ang) / W
    cy, cx = H // 2, W // 2
    rh, rw = int(rate * cy), int(rate * cx)
    cols = np.arange(W)
    col_keep = (~((cols >= cx - rw) & (cols < cx + rw))).astype(np.float64)
    A = np.real((Wc * col_keep[None, :]) @ Vc).astype(np.float32)
    return A, cy - rh, cy + rh


def _row_filter_body(TM, H, lo, hi, x_ref, a_ref, o_ref):
    x = x_ref[...]
    y = jnp.dot(x.astype(jnp.bfloat16), a_ref[...],
                preferred_element_type=jnp.float32)
    o_ref[...] = y
    for base in range(0, TM, H):
        if lo > 0:
            o_ref[base:base + lo, :] = x[base:base + lo, :]
        if hi < H:
            o_ref[base + hi:base + H, :] = x[base + hi:base + H, :]


def kernel(x, rate: float = 0.95):
    B, C, H, W = x.shape
    A_np, lo, hi = _filter_consts(int(H), int(W), float(rate))
    A = jnp.asarray(A_np, dtype=jnp.bfloat16)

    M = B * C * H
    xf = x.reshape(M, W).astype(jnp.float32)

    TM = 16384
    while M % TM != 0 or TM % H != 0:
        TM //= 2

    out = pl.pallas_call(
        functools.partial(_row_filter_body, TM, H, lo, hi),
        out_shape=jax.ShapeDtypeStruct((M, W), jnp.float32),
        grid=(M // TM,),
        in_specs=[
            pl.BlockSpec((TM, W), lambda i: (i, 0)),
            pl.BlockSpec((W, W), lambda i: (0, 0)),
        ],
        out_specs=pl.BlockSpec((TM, W), lambda i: (i, 0)),
        compiler_params=pltpu.CompilerParams(
            dimension_semantics=("arbitrary",),
            vmem_limit_bytes=64 * 2 ** 20),
    )(xf, A)

    return out.reshape(B, C, H, W)
